# baseline (speedup 1.0000x reference)
import functools

import jax
import jax.numpy as jnp
from jax import lax
from jax.experimental import pallas as pl
from jax.experimental.pallas import tpu as pltpu
from jax.experimental.pallas import tpu_sc as plsc

_V = 1000000
_D = 32
_PK = 4
_TB = 512
_PV = _V // _PK


def _mesh_info():
    info = plsc.get_sparse_core_info()
    return info.num_cores, info.num_subcores


@functools.lru_cache(maxsize=None)
def _make_transpose():
    NC, NS = _mesh_info()
    NW = NC * NS
    n_full = _V // _TB
    n_loop = (n_full // NW) * NW
    steps = n_loop // NW
    rem_rows = _V - n_full * _TB
    PR = _TB // _PK

    mesh = plsc.VectorSubcoreMesh(core_axis_name="c", subcore_axis_name="s")

    @functools.partial(
        pl.kernel,
        mesh=mesh,
        compiler_params=pltpu.CompilerParams(needs_layout_passes=False),
        out_type=jax.ShapeDtypeStruct((_PV, _PK * _D), jnp.float32),
        scratch_types=[
            pltpu.VMEM((2, _D, _TB), jnp.float32),
            pltpu.VMEM((2, PR, _PK * _D), jnp.float32),
            pltpu.SemaphoreType.DMA,
            pltpu.SemaphoreType.DMA,
            pltpu.SemaphoreType.DMA,
            pltpu.SemaphoreType.DMA,
        ],
    )
    def transpose_kernel(tableT, tail128, packed, src_v, dst_v,
                         si0, si1, so0, so1):
        wid = lax.axis_index("s") * NC + lax.axis_index("c")
        sem_i, sem_o = (si0, si1), (so0, so1)
        iota = lax.broadcasted_iota(jnp.int32, (16,), 0)

        def in_cp(blk, b, w):
            return pltpu.make_async_copy(
                tableT.at[:, pl.ds(pl.multiple_of(blk * _TB, _TB), w)],
                src_v.at[b, :, pl.ds(0, w)], sem_i[b])

        def out_cp(blk, b, w):
            return pltpu.make_async_copy(
                dst_v.at[b, pl.ds(0, w // _PK)],
                packed.at[pl.ds(pl.multiple_of(blk * PR, PR), w // _PK)],
                sem_o[b])

        def transpose_block(b, w):
            drows = [h * 4 + lax.shift_right_logical(iota, 2)
                     for h in range(8)]
            qlane = lax.bitwise_and(iota, 3)

            def pr_body(pr2, carry):
                for u in range(2):
                    pr = pr2 * 2 + u
                    col = qlane + pr * _PK
                    for h in range(8):
                        vals = plsc.load_gather(src_v.at[b], [drows[h], col])
                        dst_v[b, pr, pl.ds(h * 16, 16)] = vals
                return carry

            lax.fori_loop(0, w // _PK // 2, pr_body, 0)

        def blk_of(i):
            return wid + i * NW

        in_cp(blk_of(0), 0, _TB).start()
        in_cp(blk_of(1), 1, _TB).start()

        def body(i2, carry):
            for b in range(2):
                i = 2 * i2 + b
                blk = blk_of(i)
                in_cp(blk, b, _TB).wait()

                @pl.when(i2 >= 1)
                def _():
                    out_cp(blk_of(i - 2), b, _TB).wait()

                transpose_block(b, _TB)
                out_cp(blk, b, _TB).start()

                start_ok = (i2 <= steps // 2 - 2) if b == 1 else (
                    i2 <= steps // 2 - 1)
                @pl.when(jnp.logical_and(start_ok, True))
                def _():
                    in_cp(blk_of(i + 2), b, _TB).start()
            return carry

        n_pairs = steps // 2
        lax.fori_loop(0, n_pairs, body, 0)

        i_last = steps - 1
        in_cp(blk_of(i_last), 0, _TB).wait()
        out_cp(blk_of(i_last - 2), 0, _TB).wait()
        transpose_block(0, _TB)
        out_cp(blk_of(i_last), 0, _TB).start()

        out_cp(blk_of(i_last - 1), 1, _TB).wait()
        out_cp(blk_of(i_last), 0, _TB).wait()

        @pl.when(wid == 0)
        def _():
            pltpu.sync_copy(tableT.at[:, pl.ds(n_loop * _TB, _TB)],
                            src_v.at[0])
            transpose_block(0, _TB)
            pltpu.sync_copy(dst_v.at[0], packed.at[pl.ds(n_loop * PR, PR)])

        @pl.when(wid == 1)
        def _():
            pltpu.sync_copy(tail128, src_v.at[1, :, pl.ds(0, 128)])
            transpose_block(1, 128)
            pltpu.sync_copy(dst_v.at[1, pl.ds(0, rem_rows // _PK)],
                            packed.at[pl.ds(n_full * PR, rem_rows // _PK)])

    return transpose_kernel


@functools.lru_cache(maxsize=None)
def _make_gather(S: int, B: int):
    NC, NS = _mesh_info()
    NW = NC * NS
    cols_per_w = B // NW
    n_cb = cols_per_w // 128
    n_sg = S // 8
    s_rem = S - n_sg * 8
    n_blocks = n_sg * n_cb

    mesh = plsc.VectorSubcoreMesh(core_axis_name="c", subcore_axis_name="s")

    @functools.partial(
        pl.kernel,
        mesh=mesh,
        compiler_params=pltpu.CompilerParams(needs_layout_passes=False),
        out_type=jax.ShapeDtypeStruct((S, _D, B), jnp.float32),
        scratch_types=[
            pltpu.VMEM((8, 128), jnp.int32),
            pltpu.VMEM((8, 128), jnp.int32),
            pltpu.VMEM((8, 128), jnp.int32),
            pltpu.VMEM((2, 128, _PK * _D), jnp.float32),
            pltpu.VMEM((2, _D, 128), jnp.float32),
            pltpu.SemaphoreType.DMA,
            pltpu.SemaphoreType.DMA,
            pltpu.SemaphoreType.DMA,
            pltpu.SemaphoreType.DMA,
        ],
    )
    def gather_kernel(xT, packed, outp, idx_v, pidx_v, off_v, g_v, t_v,
                      sg0, sg1, so0, so1):
        wid = lax.axis_index("s") * NC + lax.axis_index("c")
        col0 = wid * cols_per_w
        sem_g, sem_o = (sg0, sg1), (so0, so1)
        iota = lax.broadcasted_iota(jnp.int32, (16,), 0)

        def prep_row(j):
            for v in range(8):
                xv = idx_v[j, pl.ds(16 * v, 16)]
                pidx_v[j, pl.ds(16 * v, 16)] = lax.shift_right_logical(xv, 2)
                off_v[j, pl.ds(16 * v, 16)] = lax.bitwise_and(xv, 3)

        def g_cp(j, b):
            return pltpu.make_async_copy(
                packed.at[pidx_v.at[j]], g_v.at[b], sem_g[b])

        def o_cp(plane, gc, b):
            return pltpu.make_async_copy(
                t_v.at[b], outp.at[plane, :, pl.ds(gc, 128)], sem_o[b])

        def transpose_row(j, b):
            for v in range(8):
                rows = iota + (16 * v)
                offs = off_v[j, pl.ds(16 * v, 16)]

                def d_body(d2, carry):
                    for u in range(2):
                        d = d2 * 2 + u
                        vals = plsc.load_gather(g_v.at[b],
                                                [rows, offs + d * 4])
                        t_v[b, d, pl.ds(16 * v, 16)] = vals
                    return carry

                lax.fori_loop(0, _D // 2, d_body, 0)

        def body(bid, carry):
            s0 = pl.multiple_of((bid // n_cb) * 8, 8)
            gc = pl.multiple_of(col0 + (bid % n_cb) * 128, 128)
            pltpu.sync_copy(xT.at[pl.ds(s0, 8), pl.ds(gc, 128)], idx_v)
            for j in range(8):
                prep_row(j)
            g_cp(0, 0).start()
            for j in range(8):
                b = j % 2
                if j + 1 < 8:
                    g_cp(j + 1, 1 - b).start()
                g_cp(j, b).wait()
                if j >= 2:
                    o_cp(s0 + j - 2, gc, b).wait()
                transpose_row(j, b)
                o_cp(s0 + j, gc, b).start()
            o_cp(s0 + 6, gc, 0).wait()
            o_cp(s0 + 7, gc, 1).wait()
            return carry

        lax.fori_loop(0, n_blocks, body, 0)

        s0 = n_sg * 8
        for c in range(n_cb):
            gc = col0 + c * 128
            pltpu.sync_copy(xT.at[pl.ds(s0, s_rem), pl.ds(gc, 128)],
                            idx_v.at[pl.ds(0, s_rem)])
            for j in range(s_rem):
                prep_row(j)
            for j in range(s_rem):
                g_cp(j, 0).start()
                g_cp(j, 0).wait()
                transpose_row(j, 0)
                o_cp(s0 + j, gc, 0).start()
                o_cp(s0 + j, gc, 0).wait()

    return gather_kernel


def kernel(x, table):
    S = x.shape[1]
    B = x.shape[0]
    tableT = table.T
    n_full = _V // _TB
    tail128 = jnp.pad(table[n_full * _TB:].T,
                      ((0, 0), (0, 128 - (_V - n_full * _TB))))
    packed = _make_transpose()(tableT, tail128)
    xT = x.T.astype(jnp.int32)
    outp = _make_gather(S, B)(xT, packed)
    return outp.transpose(2, 0, 1)


# speedup vs baseline: 1.0039x; 1.0039x over previous
import functools

import jax
import jax.numpy as jnp
from jax import lax
from jax.experimental import pallas as pl
from jax.experimental.pallas import tpu as pltpu
from jax.experimental.pallas import tpu_sc as plsc

_V = 1000000
_D = 32
_PK = 4
_TB = 512
_PV = _V // _PK


def _mesh_info():
    info = plsc.get_sparse_core_info()
    return info.num_cores, info.num_subcores


@functools.lru_cache(maxsize=None)
def _make_transpose():
    NC, NS = _mesh_info()
    NW = NC * NS
    n_full = _V // _TB
    n_loop = (n_full // NW) * NW
    steps = n_loop // NW
    rem_rows = _V - n_full * _TB
    PR = _TB // _PK

    mesh = plsc.VectorSubcoreMesh(core_axis_name="c", subcore_axis_name="s")

    @functools.partial(
        pl.kernel,
        mesh=mesh,
        compiler_params=pltpu.CompilerParams(needs_layout_passes=False),
        out_type=jax.ShapeDtypeStruct((_PV, _PK * _D), jnp.float32),
        scratch_types=[
            pltpu.VMEM((2, _D, _TB), jnp.float32),
            pltpu.VMEM((2, PR, _PK * _D), jnp.float32),
            pltpu.SemaphoreType.DMA,
            pltpu.SemaphoreType.DMA,
            pltpu.SemaphoreType.DMA,
            pltpu.SemaphoreType.DMA,
        ],
    )
    def transpose_kernel(tableT, tail128, packed, src_v, dst_v,
                         si0, si1, so0, so1):
        wid = lax.axis_index("s") * NC + lax.axis_index("c")
        sem_i, sem_o = (si0, si1), (so0, so1)
        iota = lax.broadcasted_iota(jnp.int32, (16,), 0)

        def in_cp(blk, b, w):
            return pltpu.make_async_copy(
                tableT.at[:, pl.ds(pl.multiple_of(blk * _TB, _TB), w)],
                src_v.at[b, :, pl.ds(0, w)], sem_i[b])

        def out_cp(blk, b, w):
            return pltpu.make_async_copy(
                dst_v.at[b, pl.ds(0, w // _PK)],
                packed.at[pl.ds(pl.multiple_of(blk * PR, PR), w // _PK)],
                sem_o[b])

        def transpose_block(b, w):
            drows = [h * 4 + lax.shift_right_logical(iota, 2)
                     for h in range(8)]
            qlane = lax.bitwise_and(iota, 3)

            def pr_body(pr, carry):
                col = qlane + pr * _PK
                for h in range(8):
                    vals = plsc.load_gather(src_v.at[b], [drows[h], col])
                    dst_v[b, pr, pl.ds(h * 16, 16)] = vals
                return carry

            lax.fori_loop(0, w // _PK, pr_body, 0)

        def blk_of(i):
            return wid + i * NW

        in_cp(blk_of(0), 0, _TB).start()
        in_cp(blk_of(1), 1, _TB).start()

        def body(i2, carry):
            for b in range(2):
                i = 2 * i2 + b
                blk = blk_of(i)
                in_cp(blk, b, _TB).wait()

                @pl.when(i2 >= 1)
                def _():
                    out_cp(blk_of(i - 2), b, _TB).wait()

                transpose_block(b, _TB)
                out_cp(blk, b, _TB).start()

                start_ok = (i2 <= steps // 2 - 2) if b == 1 else (
                    i2 <= steps // 2 - 1)
                @pl.when(jnp.logical_and(start_ok, True))
                def _():
                    in_cp(blk_of(i + 2), b, _TB).start()
            return carry

        n_pairs = steps // 2
        lax.fori_loop(0, n_pairs, body, 0)

        i_last = steps - 1
        in_cp(blk_of(i_last), 0, _TB).wait()
        out_cp(blk_of(i_last - 2), 0, _TB).wait()
        transpose_block(0, _TB)
        out_cp(blk_of(i_last), 0, _TB).start()

        out_cp(blk_of(i_last - 1), 1, _TB).wait()
        out_cp(blk_of(i_last), 0, _TB).wait()

        @pl.when(wid == 0)
        def _():
            pltpu.sync_copy(tableT.at[:, pl.ds(n_loop * _TB, _TB)],
                            src_v.at[0])
            transpose_block(0, _TB)
            pltpu.sync_copy(dst_v.at[0], packed.at[pl.ds(n_loop * PR, PR)])

        @pl.when(wid == 1)
        def _():
            pltpu.sync_copy(tail128, src_v.at[1, :, pl.ds(0, 128)])
            transpose_block(1, 128)
            pltpu.sync_copy(dst_v.at[1, pl.ds(0, rem_rows // _PK)],
                            packed.at[pl.ds(n_full * PR, rem_rows // _PK)])

    return transpose_kernel


@functools.lru_cache(maxsize=None)
def _make_gather(S: int, B: int):
    NC, NS = _mesh_info()
    NW = NC * NS
    cols_per_w = B // NW
    n_cb = cols_per_w // 128
    n_sg = S // 8
    s_rem = S - n_sg * 8
    n_blocks = n_sg * n_cb

    mesh = plsc.VectorSubcoreMesh(core_axis_name="c", subcore_axis_name="s")

    @functools.partial(
        pl.kernel,
        mesh=mesh,
        compiler_params=pltpu.CompilerParams(needs_layout_passes=False),
        out_type=jax.ShapeDtypeStruct((S, _D, B), jnp.float32),
        scratch_types=[
            pltpu.VMEM((8, 128), jnp.int32),
            pltpu.VMEM((8, 128), jnp.int32),
            pltpu.VMEM((8, 128), jnp.int32),
            pltpu.VMEM((2, 128, _PK * _D), jnp.float32),
            pltpu.VMEM((2, _D, 128), jnp.float32),
            pltpu.SemaphoreType.DMA,
            pltpu.SemaphoreType.DMA,
            pltpu.SemaphoreType.DMA,
            pltpu.SemaphoreType.DMA,
        ],
    )
    def gather_kernel(xT, packed, outp, idx_v, pidx_v, off_v, g_v, t_v,
                      sg0, sg1, so0, so1):
        wid = lax.axis_index("s") * NC + lax.axis_index("c")
        col0 = wid * cols_per_w
        sem_g, sem_o = (sg0, sg1), (so0, so1)
        iota = lax.broadcasted_iota(jnp.int32, (16,), 0)

        def prep_row(j):
            for v in range(8):
                xv = idx_v[j, pl.ds(16 * v, 16)]
                pidx_v[j, pl.ds(16 * v, 16)] = lax.shift_right_logical(xv, 2)
                off_v[j, pl.ds(16 * v, 16)] = lax.bitwise_and(xv, 3)

        def g_cp(j, b):
            return pltpu.make_async_copy(
                packed.at[pidx_v.at[j]], g_v.at[b], sem_g[b])

        def o_cp(plane, gc, b):
            return pltpu.make_async_copy(
                t_v.at[b], outp.at[plane, :, pl.ds(gc, 128)], sem_o[b])

        def transpose_row(j, b):
            for v in range(8):
                rows = iota + (16 * v)
                offs = off_v[j, pl.ds(16 * v, 16)]

                def d_body(d, carry):
                    vals = plsc.load_gather(g_v.at[b], [rows, offs + d * 4])
                    t_v[b, d, pl.ds(16 * v, 16)] = vals
                    return carry

                lax.fori_loop(0, _D, d_body, 0)

        def body(bid, carry):
            s0 = pl.multiple_of((bid // n_cb) * 8, 8)
            gc = pl.multiple_of(col0 + (bid % n_cb) * 128, 128)
            pltpu.sync_copy(xT.at[pl.ds(s0, 8), pl.ds(gc, 128)], idx_v)
            for j in range(8):
                prep_row(j)
            g_cp(0, 0).start()
            for j in range(8):
                b = j % 2
                if j + 1 < 8:
                    g_cp(j + 1, 1 - b).start()
                g_cp(j, b).wait()
                if j >= 2:
                    o_cp(s0 + j - 2, gc, b).wait()
                transpose_row(j, b)
                o_cp(s0 + j, gc, b).start()
            o_cp(s0 + 6, gc, 0).wait()
            o_cp(s0 + 7, gc, 1).wait()
            return carry

        lax.fori_loop(0, n_blocks, body, 0)

        s0 = n_sg * 8
        for c in range(n_cb):
            gc = col0 + c * 128
            pltpu.sync_copy(xT.at[pl.ds(s0, s_rem), pl.ds(gc, 128)],
                            idx_v.at[pl.ds(0, s_rem)])
            for j in range(s_rem):
                prep_row(j)
            for j in range(s_rem):
                g_cp(j, 0).start()
                g_cp(j, 0).wait()
                transpose_row(j, 0)
                o_cp(s0 + j, gc, 0).start()
                o_cp(s0 + j, gc, 0).wait()

    return gather_kernel


def kernel(x, table):
    S = x.shape[1]
    B = x.shape[0]
    tableT = table.T
    n_full = _V // _TB
    tail128 = jnp.pad(table[n_full * _TB:].T,
                      ((0, 0), (0, 128 - (_V - n_full * _TB))))
    packed = _make_transpose()(tableT, tail128)
    xT = x.T.astype(jnp.int32)
    outp = _make_gather(S, B)(xT, packed)
    return outp.transpose(2, 0, 1)


# speedup vs baseline: 1.4515x; 1.4459x over previous
import functools

import jax
import jax.numpy as jnp
from jax import lax
from jax.experimental import pallas as pl
from jax.experimental.pallas import tpu as pltpu
from jax.experimental.pallas import tpu_sc as plsc

_V = 1000000
_D = 32
_PK = 4
_TB = 512
_PV = _V // _PK


def _mesh_info():
    info = plsc.get_sparse_core_info()
    return info.num_cores, info.num_subcores


@functools.lru_cache(maxsize=None)
def _make_transpose():
    NC, NS = _mesh_info()
    NW = NC * NS
    n_full = _V // _TB
    n_loop = (n_full // NW) * NW
    steps = n_loop // NW
    rem_rows = _V - n_full * _TB
    PR = _TB // _PK

    mesh = plsc.VectorSubcoreMesh(core_axis_name="c", subcore_axis_name="s")

    @functools.partial(
        pl.kernel,
        mesh=mesh,
        compiler_params=pltpu.CompilerParams(needs_layout_passes=False),
        out_type=jax.ShapeDtypeStruct((_PV, _PK * _D), jnp.float32),
        scratch_types=[
            pltpu.VMEM((2, _D, _TB), jnp.float32),
            pltpu.VMEM((2, PR, _PK * _D), jnp.float32),
            pltpu.SemaphoreType.DMA,
            pltpu.SemaphoreType.DMA,
            pltpu.SemaphoreType.DMA,
            pltpu.SemaphoreType.DMA,
        ],
    )
    def transpose_kernel(tableT, tail128, packed, src_v, dst_v,
                         si0, si1, so0, so1):
        wid = lax.axis_index("s") * NC + lax.axis_index("c")
        sem_i, sem_o = (si0, si1), (so0, so1)
        iota = lax.broadcasted_iota(jnp.int32, (16,), 0)

        def in_cp(blk, b, w):
            return pltpu.make_async_copy(
                tableT.at[:, pl.ds(pl.multiple_of(blk * _TB, _TB), w)],
                src_v.at[b, :, pl.ds(0, w)], sem_i[b])

        def out_cp(blk, b, w):
            return pltpu.make_async_copy(
                dst_v.at[b, pl.ds(0, w // _PK)],
                packed.at[pl.ds(pl.multiple_of(blk * PR, PR), w // _PK)],
                sem_o[b])

        def transpose_block(b, w):
            lo2 = lax.shift_right_logical(iota, 2)
            q2 = lax.bitwise_and(iota, 3)
            sigma = lo2 + lax.shift_left(q2, 2)
            drs = (sigma, sigma + 16)
            cols = (lax.shift_left(sigma, 2) + q2,
                    lax.shift_left(sigma, 2) + q2 + 64)

            def m_body(m, carry):
                ic = iota + m * 16
                prv = lo2 + m * 4
                for hp in range(2):
                    vals = plsc.load_gather(src_v.at[b], [drs[hp], ic])
                    plsc.store_scatter(dst_v.at[b], [prv, cols[hp]], vals)
                return carry

            lax.fori_loop(0, w // 16, m_body, 0)

        def blk_of(i):
            return wid + i * NW

        in_cp(blk_of(0), 0, _TB).start()
        in_cp(blk_of(1), 1, _TB).start()

        def body(i2, carry):
            for b in range(2):
                i = 2 * i2 + b
                blk = blk_of(i)
                in_cp(blk, b, _TB).wait()

                @pl.when(i2 >= 1)
                def _():
                    out_cp(blk_of(i - 2), b, _TB).wait()

                transpose_block(b, _TB)
                out_cp(blk, b, _TB).start()

                start_ok = (i2 <= steps // 2 - 2) if b == 1 else (
                    i2 <= steps // 2 - 1)
                @pl.when(jnp.logical_and(start_ok, True))
                def _():
                    in_cp(blk_of(i + 2), b, _TB).start()
            return carry

        n_pairs = steps // 2
        lax.fori_loop(0, n_pairs, body, 0)

        i_last = steps - 1
        in_cp(blk_of(i_last), 0, _TB).wait()
        out_cp(blk_of(i_last - 2), 0, _TB).wait()
        transpose_block(0, _TB)
        out_cp(blk_of(i_last), 0, _TB).start()

        out_cp(blk_of(i_last - 1), 1, _TB).wait()
        out_cp(blk_of(i_last), 0, _TB).wait()

        @pl.when(wid == 0)
        def _():
            pltpu.sync_copy(tableT.at[:, pl.ds(n_loop * _TB, _TB)],
                            src_v.at[0])
            transpose_block(0, _TB)
            pltpu.sync_copy(dst_v.at[0], packed.at[pl.ds(n_loop * PR, PR)])

        @pl.when(wid == 1)
        def _():
            pltpu.sync_copy(tail128, src_v.at[1, :, pl.ds(0, 128)])
            transpose_block(1, 128)
            pltpu.sync_copy(dst_v.at[1, pl.ds(0, rem_rows // _PK)],
                            packed.at[pl.ds(n_full * PR, rem_rows // _PK)])

    return transpose_kernel


@functools.lru_cache(maxsize=None)
def _make_gather(S: int, B: int):
    NC, NS = _mesh_info()
    NW = NC * NS
    cols_per_w = B // NW
    n_cb = cols_per_w // 128
    n_sg = S // 8
    s_rem = S - n_sg * 8
    n_blocks = n_sg * n_cb

    mesh = plsc.VectorSubcoreMesh(core_axis_name="c", subcore_axis_name="s")

    @functools.partial(
        pl.kernel,
        mesh=mesh,
        compiler_params=pltpu.CompilerParams(needs_layout_passes=False),
        out_type=jax.ShapeDtypeStruct((S, _D, B), jnp.float32),
        scratch_types=[
            pltpu.VMEM((8, 128), jnp.int32),
            pltpu.VMEM((8, 128), jnp.int32),
            pltpu.VMEM((8, 128), jnp.int32),
            pltpu.VMEM((2, 128, _PK * _D), jnp.float32),
            pltpu.VMEM((2, _D, 128), jnp.float32),
            pltpu.SemaphoreType.DMA,
            pltpu.SemaphoreType.DMA,
            pltpu.SemaphoreType.DMA,
            pltpu.SemaphoreType.DMA,
        ],
    )
    def gather_kernel(xT, packed, outp, idx_v, pidx_v, off_v, g_v, t_v,
                      sg0, sg1, so0, so1):
        wid = lax.axis_index("s") * NC + lax.axis_index("c")
        col0 = wid * cols_per_w
        sem_g, sem_o = (sg0, sg1), (so0, so1)
        iota = lax.broadcasted_iota(jnp.int32, (16,), 0)

        def prep_row(j):
            for v in range(8):
                xv = idx_v[j, pl.ds(16 * v, 16)]
                pidx_v[j, pl.ds(16 * v, 16)] = lax.shift_right_logical(xv, 2)
                off_v[j, pl.ds(16 * v, 16)] = lax.bitwise_and(xv, 3)

        def g_cp(j, b):
            return pltpu.make_async_copy(
                packed.at[pidx_v.at[j]], g_v.at[b], sem_g[b])

        def o_cp(plane, gc, b):
            return pltpu.make_async_copy(
                t_v.at[b], outp.at[plane, :, pl.ds(gc, 128)], sem_o[b])

        def transpose_row(j, b):
            for v in range(8):
                rows = iota + (16 * v)
                offs = off_v[j, pl.ds(16 * v, 16)]

                def d_body(d, carry):
                    vals = plsc.load_gather(g_v.at[b], [rows, offs + d * 4])
                    t_v[b, d, pl.ds(16 * v, 16)] = vals
                    return carry

                lax.fori_loop(0, _D, d_body, 0)

        def body(bid, carry):
            s0 = pl.multiple_of((bid // n_cb) * 8, 8)
            gc = pl.multiple_of(col0 + (bid % n_cb) * 128, 128)
            pltpu.sync_copy(xT.at[pl.ds(s0, 8), pl.ds(gc, 128)], idx_v)
            for j in range(8):
                prep_row(j)
            g_cp(0, 0).start()
            for j in range(8):
                b = j % 2
                if j + 1 < 8:
                    g_cp(j + 1, 1 - b).start()
                g_cp(j, b).wait()
                if j >= 2:
                    o_cp(s0 + j - 2, gc, b).wait()
                transpose_row(j, b)
                o_cp(s0 + j, gc, b).start()
            o_cp(s0 + 6, gc, 0).wait()
            o_cp(s0 + 7, gc, 1).wait()
            return carry

        lax.fori_loop(0, n_blocks, body, 0)

        s0 = n_sg * 8
        for c in range(n_cb):
            gc = col0 + c * 128
            pltpu.sync_copy(xT.at[pl.ds(s0, s_rem), pl.ds(gc, 128)],
                            idx_v.at[pl.ds(0, s_rem)])
            for j in range(s_rem):
                prep_row(j)
            for j in range(s_rem):
                g_cp(j, 0).start()
                g_cp(j, 0).wait()
                transpose_row(j, 0)
                o_cp(s0 + j, gc, 0).start()
                o_cp(s0 + j, gc, 0).wait()

    return gather_kernel


def kernel(x, table):
    S = x.shape[1]
    B = x.shape[0]
    tableT = table.T
    n_full = _V // _TB
    tail128 = jnp.pad(table[n_full * _TB:].T,
                      ((0, 0), (0, 128 - (_V - n_full * _TB))))
    packed = _make_transpose()(tableT, tail128)
    xT = x.T.astype(jnp.int32)
    outp = _make_gather(S, B)(xT, packed)
    return outp.transpose(2, 0, 1)
